# baseline (speedup 1.0000x reference)
import functools

import jax
import jax.numpy as jnp
from jax.experimental import pallas as pl
from jax.experimental.pallas import tpu as pltpu

_CONF_THR = 0.05
_NMS_THR = 0.5
_SELECT_TOPK = 1000
_NMS_KEEP = 100
_NEG = -1e9

_KP = 1024
_BK = 128
_NBLK = _KP // _BK


def _nms_kernel(p_ref, keep_ref, sup_ref):
    P = p_ref[0]
    x1 = P[0:1, :] * 512.0
    y1 = P[1:2, :] * 512.0
    w = jnp.abs(P[2:3, :]) * 64.0 + 1.0
    h = jnp.abs(P[3:4, :]) * 64.0 + 1.0
    x2 = x1 + w
    y2 = y1 + h
    lab = P[4:5, :]
    sc = P[5:6, :]

    validf = jnp.where(sc > (_NEG / 2.0), 1.0, 0.0)

    mx = jnp.maximum(jnp.maximum(x1 * validf, y1 * validf),
                     jnp.maximum(x2 * validf, y2 * validf))
    max_coord = jnp.max(mx, axis=1, keepdims=True)

    off = lab * (max_coord + 1.0)
    x1o = x1 + off
    y1o = y1 + off
    x2o = x2 + off
    y2o = y2 + off
    area = (x2o - x1o) * (y2o - y1o)

    zero = jnp.zeros_like(x1o)
    Q8 = jnp.concatenate([x1o, y1o, x2o, y2o, area, zero, zero, zero], axis=0)
    QT = Q8.T

    for rb in range(_NBLK):
        lo = rb * _BK
        QTb = QT[lo:lo + _BK, :]
        x1c = QTb[:, 0:1]
        y1c = QTb[:, 1:2]
        x2c = QTb[:, 2:3]
        y2c = QTb[:, 3:4]
        ac = QTb[:, 4:5]
        ltx = jnp.maximum(x1c, x1o)
        lty = jnp.maximum(y1c, y1o)
        rbx = jnp.minimum(x2c, x2o)
        rby = jnp.minimum(y2c, y2o)
        wx = jnp.maximum(rbx - ltx, 0.0)
        wy = jnp.maximum(rby - lty, 0.0)
        inter = wx * wy
        union = ac + area - inter
        iou = inter / (union + 1e-9)
        jidx = jax.lax.broadcasted_iota(jnp.int32, (_BK, _KP), 1)
        iidx = jax.lax.broadcasted_iota(jnp.int32, (_BK, _KP), 0) + lo
        supf = jnp.where((iou > _NMS_THR) & (jidx > iidx), 1.0, 0.0)
        sup_ref[lo:lo + _BK, :] = supf

    kb = validf
    for g in range(_KP // 8):
        base = g * 8
        rows8 = sup_ref[base:base + 8, :]
        kv = [kb[:, base + r:base + r + 1] for r in range(8)]
        for rr in range(1, 8):
            s = kv[0] * rows8[0:1, base + rr:base + rr + 1]
            for r in range(1, rr):
                s = s + kv[r] * rows8[r:r + 1, base + rr:base + rr + 1]
            kv[rr] = kv[rr] * (1.0 - jnp.minimum(s, 1.0))
        sv = kv[0] * rows8[0:1, :]
        for r in range(1, 8):
            sv = sv + kv[r] * rows8[r:r + 1, :]
        kb = kb * (1.0 - jnp.minimum(sv, 1.0))

    keep_ref[0] = kb


def _run_nms(packed):
    B = packed.shape[0]
    return pl.pallas_call(
        _nms_kernel,
        grid=(B,),
        in_specs=[pl.BlockSpec((1, 8, _KP), lambda i: (i, 0, 0))],
        out_specs=pl.BlockSpec((1, 1, _KP), lambda i: (i, 0, 0)),
        out_shape=jax.ShapeDtypeStruct((B, 1, _KP), jnp.float32),
        scratch_shapes=[
            pltpu.VMEM((_KP, _KP), jnp.float32),
        ],
        compiler_params=pltpu.CompilerParams(
            dimension_semantics=("parallel",)),
    )(packed)


@jax.jit
def kernel(pscores, pboxes, plabels):
    B, N = pscores.shape
    scores_m = jnp.where(pscores > _CONF_THR, pscores, _NEG)
    top_s, top_i = jax.lax.top_k(scores_m, _SELECT_TOPK)
    top_praw = jnp.take_along_axis(pboxes, top_i[..., None], axis=1)
    top_l = jnp.take_along_axis(plabels, top_i, axis=1)

    pad = _KP - _SELECT_TOPK
    coords = jnp.moveaxis(top_praw, 2, 1)
    rows = jnp.concatenate([
        coords,
        top_l[:, None, :].astype(jnp.float32),
        top_s[:, None, :],
        jnp.zeros((B, 2, _SELECT_TOPK), jnp.float32),
    ], axis=1)
    packed = jnp.pad(rows, ((0, 0), (0, 0), (0, pad)),
                     constant_values=0.0)
    packed = packed.at[:, 5, _SELECT_TOPK:].set(_NEG)

    keepf = _run_nms(packed)[:, 0, :_SELECT_TOPK]
    keep = keepf > 0.5

    kscores = jnp.where(keep, top_s, _NEG)
    out_s, sel = jax.lax.top_k(kscores, _NMS_KEEP)

    lt = top_praw[..., :2] * 512.0
    wh = jnp.abs(top_praw[..., 2:]) * 64.0 + 1.0
    top_boxes = jnp.concatenate([lt, lt + wh], axis=-1)

    out_b = jnp.take_along_axis(top_boxes, sel[..., None], axis=1)
    out_l = jnp.take_along_axis(top_l, sel, axis=1)
    ids_batch = jnp.broadcast_to(jnp.arange(B, dtype=top_l.dtype)[:, None],
                                 out_s.shape)
    return ids_batch, out_b, out_l, out_s


# speedup vs baseline: 1.2882x; 1.2882x over previous
import functools

import jax
import jax.numpy as jnp
from jax.experimental import pallas as pl
from jax.experimental.pallas import tpu as pltpu

_CONF_THR = 0.05
_NMS_THR = 0.5
_SELECT_TOPK = 1000
_NMS_KEEP = 100
_NEG = -1e9

_KP = 1024
_BK = 128
_NBLK = _KP // _BK


def _nms_kernel(p_ref, keep_ref, sup_ref, kscr_ref):
    P = p_ref[0]
    x1 = P[0:1, :] * 512.0
    y1 = P[1:2, :] * 512.0
    w = jnp.abs(P[2:3, :]) * 64.0 + 1.0
    h = jnp.abs(P[3:4, :]) * 64.0 + 1.0
    x2 = x1 + w
    y2 = y1 + h
    lab = P[4:5, :]
    sc = P[5:6, :]

    validf = jnp.where(sc > (_NEG / 2.0), 1.0, 0.0)

    mx = jnp.maximum(jnp.maximum(x1 * validf, y1 * validf),
                     jnp.maximum(x2 * validf, y2 * validf))
    max_coord = jnp.max(mx, axis=1, keepdims=True)

    off = lab * (max_coord + 1.0)
    x1o = x1 + off
    y1o = y1 + off
    x2o = x2 + off
    y2o = y2 + off
    area = (x2o - x1o) * (y2o - y1o)

    zero = jnp.zeros_like(x1o)
    Q8 = jnp.concatenate([x1o, y1o, x2o, y2o, area, zero, zero, zero], axis=0)
    QT = Q8.T

    for rb in range(_NBLK):
        lo = rb * _BK
        QTb = QT[lo:lo + _BK, :]
        x1c = QTb[:, 0:1]
        y1c = QTb[:, 1:2]
        x2c = QTb[:, 2:3]
        y2c = QTb[:, 3:4]
        ac = QTb[:, 4:5]
        ltx = jnp.maximum(x1c, x1o)
        lty = jnp.maximum(y1c, y1o)
        rbx = jnp.minimum(x2c, x2o)
        rby = jnp.minimum(y2c, y2o)
        wx = jnp.maximum(rbx - ltx, 0.0)
        wy = jnp.maximum(rby - lty, 0.0)
        inter = wx * wy
        union = ac + area - inter
        iou = inter / (union + 1e-9)
        jidx = jax.lax.broadcasted_iota(jnp.int32, (_BK, _KP), 1)
        iidx = jax.lax.broadcasted_iota(jnp.int32, (_BK, _KP), 0) + lo
        supf = jnp.where((iou > _NMS_THR) & (jidx > iidx), 1.0, 0.0)
        sup_ref[lo:lo + _BK, :] = supf

    kscr_ref[0:1, :] = validf
    lidx = jax.lax.broadcasted_iota(jnp.int32, (1, _BK), 1)

    for b in range(_NBLK):
        lo = b * _BK
        kb0 = kscr_ref[0:1, lo:lo + _BK]

        def body(g, kb, lo=lo):
            base = pl.multiple_of(lo + g * 8, 8)
            rows8 = sup_ref[pl.ds(base, 8), lo:lo + _BK]
            for r in range(8):
                i = g * 8 + r
                row = rows8[r:r + 1, :]
                ki = jnp.sum(kb * jnp.where(lidx == i, 1.0, 0.0),
                             axis=1, keepdims=True)
                kb = kb * (1.0 - ki * row)
            return kb

        kb = jax.lax.fori_loop(0, _BK // 8, body, kb0)
        kscr_ref[0:1, lo:lo + _BK] = kb
        SUPb = sup_ref[lo:lo + _BK, :]
        lat = jax.lax.dot_general(
            kb, SUPb, (((1,), (0,)), ((), ())),
            preferred_element_type=jnp.float32,
            precision=jax.lax.Precision.HIGHEST)
        kscr_ref[0:1, :] = kscr_ref[0:1, :] * (1.0 - jnp.minimum(lat, 1.0))

    keep_ref[0] = kscr_ref[0:1, :]


def _run_nms(packed):
    B = packed.shape[0]
    return pl.pallas_call(
        _nms_kernel,
        grid=(B,),
        in_specs=[pl.BlockSpec((1, 8, _KP), lambda i: (i, 0, 0))],
        out_specs=pl.BlockSpec((1, 1, _KP), lambda i: (i, 0, 0)),
        out_shape=jax.ShapeDtypeStruct((B, 1, _KP), jnp.float32),
        scratch_shapes=[
            pltpu.VMEM((_KP, _KP), jnp.float32),
            pltpu.VMEM((8, _KP), jnp.float32),
        ],
    )(packed)


@jax.jit
def kernel(pscores, pboxes, plabels):
    B, N = pscores.shape
    scores_m = jnp.where(pscores > _CONF_THR, pscores, _NEG)
    top_s, top_i = jax.lax.top_k(scores_m, _SELECT_TOPK)
    top_praw = jnp.take_along_axis(pboxes, top_i[..., None], axis=1)
    top_l = jnp.take_along_axis(plabels, top_i, axis=1)

    pad = _KP - _SELECT_TOPK
    coords = jnp.moveaxis(top_praw, 2, 1)
    rows = jnp.concatenate([
        coords,
        top_l[:, None, :].astype(jnp.float32),
        top_s[:, None, :],
        jnp.zeros((B, 2, _SELECT_TOPK), jnp.float32),
    ], axis=1)
    packed = jnp.pad(rows, ((0, 0), (0, 0), (0, pad)),
                     constant_values=0.0)
    packed = packed.at[:, 5, _SELECT_TOPK:].set(_NEG)

    keepf = _run_nms(packed)[:, 0, :_SELECT_TOPK]
    keep = keepf > 0.5

    kscores = jnp.where(keep, top_s, _NEG)
    out_s, sel = jax.lax.top_k(kscores, _NMS_KEEP)

    lt = top_praw[..., :2] * 512.0
    wh = jnp.abs(top_praw[..., 2:]) * 64.0 + 1.0
    top_boxes = jnp.concatenate([lt, lt + wh], axis=-1)

    out_b = jnp.take_along_axis(top_boxes, sel[..., None], axis=1)
    out_l = jnp.take_along_axis(top_l, sel, axis=1)
    ids_batch = jnp.broadcast_to(jnp.arange(B, dtype=top_l.dtype)[:, None],
                                 out_s.shape)
    return ids_batch, out_b, out_l, out_s
